# Initial kernel scaffold; baseline (speedup 1.0000x reference)
import jax
import jax.numpy as jnp
from jax.experimental import pallas as pl


def kernel(x, edge_index, batch, W1, b1, W2, b2, W3, b3, Wmu, bmu, Wlv, blv, Wf1, bf1, Wf2, bf2):
    raise NotImplementedError("write your pallas kernel here")


# speedup vs baseline: 12.3317x; 12.3317x over previous
import functools

import jax
import jax.numpy as jnp
from jax import lax
from jax.experimental import pallas as pl
from jax.experimental.pallas import tpu as pltpu
from jax.experimental.pallas import tpu_sc as plsc

NC = 2
NS = 16
CC = 128
EC = 80
RB = 64
ROW_BLK = 1024


def _sc_mesh():
  return plsc.VectorSubcoreMesh(core_axis_name="c", subcore_axis_name="s",
                                num_cores=NC, num_subcores=NS)


IB = 25


@functools.cache
def _make_agg(n, e):
  ept = e // NS
  nw = ept // (IB * EC)
  nwc = nw // NC
  rpt = n // NS
  nr = rpt // RB

  out_type = [jax.ShapeDtypeStruct((n, CC), jnp.float32) for _ in range(2)]
  scratch = [
      pltpu.VMEM((IB, EC), jnp.int32),
      pltpu.VMEM((IB, EC), jnp.int32),
      pltpu.VMEM((EC, CC), jnp.float32),
      pltpu.VMEM((RB, CC), jnp.float32),
      pltpu.VMEM_SHARED((n, CC), jnp.float32),
      pltpu.SemaphoreType.DMA,
  ]

  @functools.partial(pl.kernel, out_type=out_type, mesh=_sc_mesh(),
                     scratch_types=scratch,
                     compiler_params=pltpu.CompilerParams(
                         use_tc_tiling_on_sc=False))
  def agg(src_hbm, dst_hbm, zeros_hbm, xs, out0, out1,
          src_v, dst_v, rows_v, bounce, accum, sem):
    cid = lax.axis_index("c")
    tid = lax.axis_index("s")
    row0 = tid * rpt
    for c in range(NC):
      out = out0 if c == 0 else out1

      @pl.when(cid == c)
      def _(c=c, out=out):
        for r in range(nr):
          if c == 0:
            pltpu.sync_copy(xs.at[pl.ds(row0 + r * RB, RB)], bounce)
          else:
            pltpu.sync_copy(zeros_hbm, bounce)
          pltpu.sync_copy(bounce, accum.at[pl.ds(row0 + r * RB, RB)])
        plsc.subcore_barrier()

        def outer(jo, carry):
          pltpu.sync_copy(src_hbm.at[tid, jo], src_v)
          pltpu.sync_copy(dst_hbm.at[tid, jo], dst_v)

          def body(j, carry2):
            pltpu.async_copy(xs.at[src_v.at[j]], rows_v, sem).wait()
            pltpu.sync_copy(rows_v, accum.at[dst_v.at[j]], add=True)
            return carry2

          lax.fori_loop(0, IB, body, 0)
          return carry

        lax.fori_loop(c * nwc, (c + 1) * nwc, outer, 0)
        plsc.subcore_barrier()
        for r in range(nr):
          pltpu.sync_copy(accum.at[pl.ds(row0 + r * RB, RB)], bounce)
          pltpu.sync_copy(bounce, out.at[pl.ds(row0 + r * RB, RB)])
        plsc.subcore_barrier()

  return agg


def _make_deg(n, e):
  ept = e // NS
  nw = ept // (IB * EC)
  nwc = nw // NC
  rpt = n // NS

  out_type = [jax.ShapeDtypeStruct((n, 8), jnp.float32) for _ in range(NC)]
  scratch = [
      pltpu.VMEM((IB, EC), jnp.int32),
      pltpu.VMEM((rpt, 8), jnp.float32),
      pltpu.VMEM((EC, 8), jnp.float32),
      pltpu.VMEM_SHARED((n, 8), jnp.float32),
  ]

  @functools.partial(pl.kernel, out_type=out_type, mesh=_sc_mesh(),
                     scratch_types=scratch,
                     compiler_params=pltpu.CompilerParams(
                         use_tc_tiling_on_sc=False))
  def deg(dst_hbm, ones_hbm, zeros_hbm, out0, out1, dst_v, bounce, ones_v,
          accum):
    cid = lax.axis_index("c")
    tid = lax.axis_index("s")
    row0 = tid * rpt
    pltpu.sync_copy(ones_hbm.at[pl.ds(0, EC)], ones_v)
    for c in range(NC):

      @pl.when(cid == c)
      def _(c=c):
        if c == 0:
          pltpu.sync_copy(ones_hbm, bounce)
        else:
          pltpu.sync_copy(zeros_hbm, bounce)
        pltpu.sync_copy(bounce, accum.at[pl.ds(row0, rpt)])
        plsc.subcore_barrier()

        def outer(jo, carry):
          pltpu.sync_copy(dst_hbm.at[tid, jo], dst_v)

          def body(j, carry2):
            pltpu.sync_copy(ones_v, accum.at[dst_v.at[j]], add=True)
            return carry2

          lax.fori_loop(0, IB, body, 0)
          return carry

        lax.fori_loop(c * nwc, (c + 1) * nwc, outer, 0)
        plsc.subcore_barrier()
        pltpu.sync_copy(accum.at[pl.ds(row0, rpt)], bounce)
        pltpu.sync_copy(bounce, (out0 if c == 0 else out1).at[
            pl.ds(row0, rpt)])

  return deg


def _chunk_specs(nchunks):
  return [pl.BlockSpec((ROW_BLK, CC), lambda i: (i, 0))
          for _ in range(nchunks)]


def _prep(x, deg0, deg1):
  n, d = x.shape
  nc = d // CC

  def body(x_ref, d0_ref, d1_ref, *outs):
    dinv = lax.rsqrt(d0_ref[:, :1] + d1_ref[:, :1])
    xs = x_ref[...] * dinv
    for k in range(nc):
      outs[k][...] = xs[:, k * CC:(k + 1) * CC]

  return pl.pallas_call(
      body, grid=(n // ROW_BLK,),
      in_specs=[pl.BlockSpec((ROW_BLK, d), lambda i: (i, 0)),
                pl.BlockSpec((ROW_BLK, 8), lambda i: (i, 0)),
                pl.BlockSpec((ROW_BLK, 8), lambda i: (i, 0))],
      out_specs=_chunk_specs(nc),
      out_shape=[jax.ShapeDtypeStruct((n, CC), jnp.float32)] * nc,
  )(x, deg0, deg1)


def _layer(pairs, deg0, deg1, W, b, relu, emit_raw):
  n = pairs[0].shape[0]
  win, wout = W.shape
  nci, nco = win // CC, wout // CC

  def body(*refs):
    cr = refs[:2 * nci]
    d0_ref, d1_ref, w_ref, b_ref = refs[2 * nci:2 * nci + 4]
    outs = refs[2 * nci + 4:]
    dinv = lax.rsqrt(d0_ref[:, :1] + d1_ref[:, :1])
    s = jnp.concatenate(
        [cr[2 * k][...] + cr[2 * k + 1][...] for k in range(nci)],
        axis=1) * dinv
    h = jnp.dot(s, w_ref[...], preferred_element_type=jnp.float32) + b_ref[...]
    if relu:
      h = jnp.maximum(h, 0.0)
    hs = h * dinv
    for k in range(nco):
      outs[k][...] = hs[:, k * CC:(k + 1) * CC]
    if emit_raw:
      for k in range(nco):
        outs[nco + k][...] = h[:, k * CC:(k + 1) * CC]

  nout = nco * (2 if emit_raw else 1)
  res = pl.pallas_call(
      body, grid=(n // ROW_BLK,),
      in_specs=_chunk_specs(2 * nci) + [
          pl.BlockSpec((ROW_BLK, 8), lambda i: (i, 0)),
          pl.BlockSpec((ROW_BLK, 8), lambda i: (i, 0)),
          pl.BlockSpec((win, wout), lambda i: (0, 0)),
          pl.BlockSpec((wout,), lambda i: (0,)),
      ],
      out_specs=_chunk_specs(nout),
      out_shape=[jax.ShapeDtypeStruct((n, CC), jnp.float32)] * nout,
  )(*pairs, deg0, deg1, W, b)
  if emit_raw:
    return res[:nco], res[nco:]
  return res


def _final(pairs, deg0, deg1, Wmu, bmu, Wlv, blv, eps):
  n = pairs[0].shape[0]
  win, wout = Wmu.shape
  nci = win // CC

  def body(*refs):
    cr = refs[:2 * nci]
    (d0_ref, d1_ref, wmu_ref, bmu_ref, wlv_ref, blv_ref,
     eps_ref) = refs[2 * nci:2 * nci + 7]
    z_ref, mu_ref, lv_ref = refs[2 * nci + 7:]
    dinv = lax.rsqrt(d0_ref[:, :1] + d1_ref[:, :1])
    s = jnp.concatenate(
        [cr[2 * k][...] + cr[2 * k + 1][...] for k in range(nci)],
        axis=1) * dinv
    mu = jnp.dot(s, wmu_ref[...],
                 preferred_element_type=jnp.float32) + bmu_ref[...]
    lv = jnp.dot(s, wlv_ref[...],
                 preferred_element_type=jnp.float32) + blv_ref[...]
    mu_ref[...] = mu
    lv_ref[...] = lv
    z_ref[...] = mu + eps_ref[...] * jnp.exp(0.5 * lv)

  return pl.pallas_call(
      body, grid=(n // ROW_BLK,),
      in_specs=_chunk_specs(2 * nci) + [
          pl.BlockSpec((ROW_BLK, 8), lambda i: (i, 0)),
          pl.BlockSpec((ROW_BLK, 8), lambda i: (i, 0)),
          pl.BlockSpec((win, wout), lambda i: (0, 0)),
          pl.BlockSpec((wout,), lambda i: (0,)),
          pl.BlockSpec((win, wout), lambda i: (0, 0)),
          pl.BlockSpec((wout,), lambda i: (0,)),
          pl.BlockSpec((ROW_BLK, wout), lambda i: (i, 0)),
      ],
      out_specs=[pl.BlockSpec((ROW_BLK, wout), lambda i: (i, 0))] * 3,
      out_shape=[jax.ShapeDtypeStruct((n, wout), jnp.float32)] * 3,
  )(*pairs, deg0, deg1, Wmu, bmu, Wlv, blv, eps)


def _head(xr_chunks, batch2d, Wf1, bf1, Wf2, bf2, num_graphs):
  n = xr_chunks[0].shape[0]
  nci = len(xr_chunks)
  d = nci * CC
  dh = Wf1.shape[1]
  do = Wf2.shape[1]
  nsteps = n // ROW_BLK

  def body(*refs):
    cr = refs[:nci]
    b_ref, w1_ref, b1_ref, w2_ref, b2_ref = refs[nci:nci + 5]
    out_ref = refs[nci + 5]
    acc = refs[nci + 6]
    i = pl.program_id(0)

    @pl.when(i == 0)
    def _():
      acc[...] = jnp.full((num_graphs, d), -jnp.inf, jnp.float32)

    xr = jnp.concatenate([r[...] for r in cr], axis=1)
    bid = b_ref[:, :1]
    for g in range(num_graphs):
      m = jnp.max(jnp.where(bid == g, xr, -jnp.inf), axis=0, keepdims=True)
      acc[g:g + 1, :] = jnp.maximum(acc[g:g + 1, :], m)

    @pl.when(i == nsteps - 1)
    def _():
      x2 = acc[...]
      h = jnp.maximum(
          jnp.dot(x2, w1_ref[...], preferred_element_type=jnp.float32)
          + b1_ref[...], 0.0)
      out_ref[...] = (
          jnp.dot(h, w2_ref[...], preferred_element_type=jnp.float32)
          + b2_ref[...])

  return pl.pallas_call(
      body, grid=(nsteps,),
      in_specs=_chunk_specs(nci) + [
          pl.BlockSpec((ROW_BLK, 8), lambda i: (i, 0)),
          pl.BlockSpec((d, dh), lambda i: (0, 0)),
          pl.BlockSpec((dh,), lambda i: (0,)),
          pl.BlockSpec((dh, do), lambda i: (0, 0)),
          pl.BlockSpec((do,), lambda i: (0,)),
      ],
      out_specs=pl.BlockSpec((num_graphs, do), lambda i: (0, 0)),
      out_shape=jax.ShapeDtypeStruct((num_graphs, do), jnp.float32),
      scratch_shapes=[pltpu.VMEM((num_graphs, d), jnp.float32)],
  )(*xr_chunks, batch2d, Wf1, bf1, Wf2, bf2)


def kernel(x, edge_index, batch, W1, b1, W2, b2, W3, b3, Wmu, bmu, Wlv, blv,
           Wf1, bf1, Wf2, bf2):
  n, d = x.shape
  e = edge_index.shape[1]
  num_graphs = 64
  npad = -(-n // ROW_BLK) * ROW_BLK
  npad += -npad % (NS * 8)

  xp = jnp.pad(x, ((0, npad - n), (0, 0)))
  src = edge_index[0].reshape(NS, -1, IB, EC)
  dst = edge_index[1].reshape(NS, -1, IB, EC)
  ones = jnp.ones((npad // NS, 8), jnp.float32)
  zeros8 = jnp.zeros((npad // NS, 8), jnp.float32)
  zeros = jnp.zeros((RB, CC), jnp.float32)
  batch_p = jnp.pad(batch, (0, npad - n), constant_values=num_graphs)
  batch2d = jnp.broadcast_to(batch_p[:, None], (npad, 8))
  eps = jax.random.normal(jax.random.key(42), (n, Wmu.shape[1]),
                          dtype=jnp.float32)
  eps_p = jnp.pad(eps, ((0, npad - n), (0, 0)))

  aggf = _make_agg(npad, e)

  def agg_all(chunks):
    outs = []
    for chk in chunks:
      outs.extend(aggf(src, dst, zeros, chk))
    return outs

  deg0, deg1 = _make_deg(npad, e)(dst, ones, zeros8)
  xs0 = _prep(xp, deg0, deg1)
  s0 = agg_all(xs0)
  h1 = _layer(s0, deg0, deg1, W1, b1, relu=True, emit_raw=False)
  s1 = agg_all(h1)
  h2 = _layer(s1, deg0, deg1, W2, b2, relu=True, emit_raw=False)
  s2 = agg_all(h2)
  xs3, xr = _layer(s2, deg0, deg1, W3, b3, relu=True, emit_raw=True)
  s3 = agg_all(xs3)
  z, mu, lv = _final(s3, deg0, deg1, Wmu, bmu, Wlv, blv, eps_p)
  pm = _head(xr, batch2d, Wf1, bf1, Wf2, bf2, num_graphs)
  return (z[:n], mu[:n], lv[:n], pm)


# speedup vs baseline: 14.3015x; 1.1597x over previous
import functools

import jax
import jax.numpy as jnp
from jax import lax
from jax.experimental import pallas as pl
from jax.experimental.pallas import tpu as pltpu
from jax.experimental.pallas import tpu_sc as plsc

NC = 2
NS = 16
CC = 128
EC = 80
RB = 32
ROW_BLK = 1024


def _sc_mesh():
  return plsc.VectorSubcoreMesh(core_axis_name="c", subcore_axis_name="s",
                                num_cores=NC, num_subcores=NS)


IB = 25


@functools.cache
def _make_agg(n, e):
  ept = e // NS
  nw = ept // (IB * EC)
  nwc = nw // NC
  rpt = n // NS
  nr = rpt // RB

  out_type = [jax.ShapeDtypeStruct((n, CC), jnp.float32) for _ in range(2)]
  scratch = [
      pltpu.VMEM((IB, EC), jnp.int32),
      pltpu.VMEM((IB, EC), jnp.int32),
      pltpu.VMEM((EC, CC), jnp.float32),
      pltpu.VMEM((EC, CC), jnp.float32),
      pltpu.VMEM((RB, CC), jnp.float32),
      pltpu.VMEM_SHARED((n, CC), jnp.float32),
      pltpu.SemaphoreType.DMA,
      pltpu.SemaphoreType.DMA,
  ]

  @functools.partial(pl.kernel, out_type=out_type, mesh=_sc_mesh(),
                     scratch_types=scratch,
                     compiler_params=pltpu.CompilerParams(
                         use_tc_tiling_on_sc=False))
  def agg(src_hbm, dst_hbm, zeros_hbm, xs, out0, out1,
          src_v, dst_v, rows0, rows1, bounce, accum, sem0, sem1):
    cid = lax.axis_index("c")
    tid = lax.axis_index("s")
    row0 = tid * rpt
    for c in range(NC):
      out = out0 if c == 0 else out1

      @pl.when(cid == c)
      def _(c=c, out=out):
        for r in range(nr):
          if c == 0:
            pltpu.sync_copy(xs.at[pl.ds(row0 + r * RB, RB)], bounce)
          else:
            pltpu.sync_copy(zeros_hbm, bounce)
          pltpu.sync_copy(bounce, accum.at[pl.ds(row0 + r * RB, RB)])
        plsc.subcore_barrier()

        def outer(jo, carry):
          pltpu.sync_copy(src_hbm.at[tid, jo], src_v)
          pltpu.sync_copy(dst_hbm.at[tid, jo], dst_v)

          def body(jj, carry2):
            g0 = pltpu.async_copy(xs.at[src_v.at[2 * jj]], rows0, sem0)
            g1 = pltpu.async_copy(xs.at[src_v.at[2 * jj + 1]], rows1, sem1)
            g0.wait()
            pltpu.sync_copy(rows0, accum.at[dst_v.at[2 * jj]], add=True)
            g1.wait()
            pltpu.sync_copy(rows1, accum.at[dst_v.at[2 * jj + 1]], add=True)
            return carry2

          lax.fori_loop(0, IB // 2, body, 0)
          pltpu.async_copy(xs.at[src_v.at[IB - 1]], rows0, sem0).wait()
          pltpu.sync_copy(rows0, accum.at[dst_v.at[IB - 1]], add=True)
          return carry

        lax.fori_loop(c * nwc, (c + 1) * nwc, outer, 0)
        plsc.subcore_barrier()
        for r in range(nr):
          pltpu.sync_copy(accum.at[pl.ds(row0 + r * RB, RB)], bounce)
          pltpu.sync_copy(bounce, out.at[pl.ds(row0 + r * RB, RB)])
        plsc.subcore_barrier()

  return agg


def _make_deg(n, e):
  ept = e // NS
  nw = ept // (IB * EC)
  nwc = nw // NC
  rpt = n // NS

  out_type = [jax.ShapeDtypeStruct((n, 8), jnp.float32) for _ in range(NC)]
  scratch = [
      pltpu.VMEM((IB, EC), jnp.int32),
      pltpu.VMEM((rpt, 8), jnp.float32),
      pltpu.VMEM((EC, 8), jnp.float32),
      pltpu.VMEM_SHARED((n, 8), jnp.float32),
  ]

  @functools.partial(pl.kernel, out_type=out_type, mesh=_sc_mesh(),
                     scratch_types=scratch,
                     compiler_params=pltpu.CompilerParams(
                         use_tc_tiling_on_sc=False))
  def deg(dst_hbm, ones_hbm, zeros_hbm, out0, out1, dst_v, bounce, ones_v,
          accum):
    cid = lax.axis_index("c")
    tid = lax.axis_index("s")
    row0 = tid * rpt
    pltpu.sync_copy(ones_hbm.at[pl.ds(0, EC)], ones_v)
    for c in range(NC):

      @pl.when(cid == c)
      def _(c=c):
        if c == 0:
          pltpu.sync_copy(ones_hbm, bounce)
        else:
          pltpu.sync_copy(zeros_hbm, bounce)
        pltpu.sync_copy(bounce, accum.at[pl.ds(row0, rpt)])
        plsc.subcore_barrier()

        def outer(jo, carry):
          pltpu.sync_copy(dst_hbm.at[tid, jo], dst_v)

          def body(j, carry2):
            pltpu.sync_copy(ones_v, accum.at[dst_v.at[j]], add=True)
            return carry2

          lax.fori_loop(0, IB, body, 0)
          return carry

        lax.fori_loop(c * nwc, (c + 1) * nwc, outer, 0)
        plsc.subcore_barrier()
        pltpu.sync_copy(accum.at[pl.ds(row0, rpt)], bounce)
        pltpu.sync_copy(bounce, (out0 if c == 0 else out1).at[
            pl.ds(row0, rpt)])

  return deg


def _chunk_specs(nchunks):
  return [pl.BlockSpec((ROW_BLK, CC), lambda i: (i, 0))
          for _ in range(nchunks)]


def _prep(x, deg0, deg1):
  n, d = x.shape
  nc = d // CC

  def body(x_ref, d0_ref, d1_ref, *outs):
    dinv = lax.rsqrt(d0_ref[:, :1] + d1_ref[:, :1])
    xs = x_ref[...] * dinv
    for k in range(nc):
      outs[k][...] = xs[:, k * CC:(k + 1) * CC]

  return pl.pallas_call(
      body, grid=(n // ROW_BLK,),
      in_specs=[pl.BlockSpec((ROW_BLK, d), lambda i: (i, 0)),
                pl.BlockSpec((ROW_BLK, 8), lambda i: (i, 0)),
                pl.BlockSpec((ROW_BLK, 8), lambda i: (i, 0))],
      out_specs=_chunk_specs(nc),
      out_shape=[jax.ShapeDtypeStruct((n, CC), jnp.float32)] * nc,
  )(x, deg0, deg1)


def _layer(pairs, deg0, deg1, W, b, relu, emit_raw):
  n = pairs[0].shape[0]
  win, wout = W.shape
  nci, nco = win // CC, wout // CC

  def body(*refs):
    cr = refs[:2 * nci]
    d0_ref, d1_ref, w_ref, b_ref = refs[2 * nci:2 * nci + 4]
    outs = refs[2 * nci + 4:]
    dinv = lax.rsqrt(d0_ref[:, :1] + d1_ref[:, :1])
    s = jnp.concatenate(
        [cr[2 * k][...] + cr[2 * k + 1][...] for k in range(nci)],
        axis=1) * dinv
    h = jnp.dot(s, w_ref[...], preferred_element_type=jnp.float32) + b_ref[...]
    if relu:
      h = jnp.maximum(h, 0.0)
    hs = h * dinv
    for k in range(nco):
      outs[k][...] = hs[:, k * CC:(k + 1) * CC]
    if emit_raw:
      for k in range(nco):
        outs[nco + k][...] = h[:, k * CC:(k + 1) * CC]

  nout = nco * (2 if emit_raw else 1)
  res = pl.pallas_call(
      body, grid=(n // ROW_BLK,),
      in_specs=_chunk_specs(2 * nci) + [
          pl.BlockSpec((ROW_BLK, 8), lambda i: (i, 0)),
          pl.BlockSpec((ROW_BLK, 8), lambda i: (i, 0)),
          pl.BlockSpec((win, wout), lambda i: (0, 0)),
          pl.BlockSpec((wout,), lambda i: (0,)),
      ],
      out_specs=_chunk_specs(nout),
      out_shape=[jax.ShapeDtypeStruct((n, CC), jnp.float32)] * nout,
  )(*pairs, deg0, deg1, W, b)
  if emit_raw:
    return res[:nco], res[nco:]
  return res


def _final(pairs, deg0, deg1, Wmu, bmu, Wlv, blv, eps):
  n = pairs[0].shape[0]
  win, wout = Wmu.shape
  nci = win // CC

  def body(*refs):
    cr = refs[:2 * nci]
    (d0_ref, d1_ref, wmu_ref, bmu_ref, wlv_ref, blv_ref,
     eps_ref) = refs[2 * nci:2 * nci + 7]
    z_ref, mu_ref, lv_ref = refs[2 * nci + 7:]
    dinv = lax.rsqrt(d0_ref[:, :1] + d1_ref[:, :1])
    s = jnp.concatenate(
        [cr[2 * k][...] + cr[2 * k + 1][...] for k in range(nci)],
        axis=1) * dinv
    mu = jnp.dot(s, wmu_ref[...],
                 preferred_element_type=jnp.float32) + bmu_ref[...]
    lv = jnp.dot(s, wlv_ref[...],
                 preferred_element_type=jnp.float32) + blv_ref[...]
    mu_ref[...] = mu
    lv_ref[...] = lv
    z_ref[...] = mu + eps_ref[...] * jnp.exp(0.5 * lv)

  return pl.pallas_call(
      body, grid=(n // ROW_BLK,),
      in_specs=_chunk_specs(2 * nci) + [
          pl.BlockSpec((ROW_BLK, 8), lambda i: (i, 0)),
          pl.BlockSpec((ROW_BLK, 8), lambda i: (i, 0)),
          pl.BlockSpec((win, wout), lambda i: (0, 0)),
          pl.BlockSpec((wout,), lambda i: (0,)),
          pl.BlockSpec((win, wout), lambda i: (0, 0)),
          pl.BlockSpec((wout,), lambda i: (0,)),
          pl.BlockSpec((ROW_BLK, wout), lambda i: (i, 0)),
      ],
      out_specs=[pl.BlockSpec((ROW_BLK, wout), lambda i: (i, 0))] * 3,
      out_shape=[jax.ShapeDtypeStruct((n, wout), jnp.float32)] * 3,
  )(*pairs, deg0, deg1, Wmu, bmu, Wlv, blv, eps)


def _head(xr_chunks, batch2d, Wf1, bf1, Wf2, bf2, num_graphs):
  n = xr_chunks[0].shape[0]
  nci = len(xr_chunks)
  d = nci * CC
  dh = Wf1.shape[1]
  do = Wf2.shape[1]
  nsteps = n // ROW_BLK

  def body(*refs):
    cr = refs[:nci]
    b_ref, w1_ref, b1_ref, w2_ref, b2_ref = refs[nci:nci + 5]
    out_ref = refs[nci + 5]
    acc = refs[nci + 6]
    i = pl.program_id(0)

    @pl.when(i == 0)
    def _():
      acc[...] = jnp.full((num_graphs, d), -jnp.inf, jnp.float32)

    xr = jnp.concatenate([r[...] for r in cr], axis=1)
    bid = b_ref[:, :1]
    for g in range(num_graphs):
      m = jnp.max(jnp.where(bid == g, xr, -jnp.inf), axis=0, keepdims=True)
      acc[g:g + 1, :] = jnp.maximum(acc[g:g + 1, :], m)

    @pl.when(i == nsteps - 1)
    def _():
      x2 = acc[...]
      h = jnp.maximum(
          jnp.dot(x2, w1_ref[...], preferred_element_type=jnp.float32)
          + b1_ref[...], 0.0)
      out_ref[...] = (
          jnp.dot(h, w2_ref[...], preferred_element_type=jnp.float32)
          + b2_ref[...])

  return pl.pallas_call(
      body, grid=(nsteps,),
      in_specs=_chunk_specs(nci) + [
          pl.BlockSpec((ROW_BLK, 8), lambda i: (i, 0)),
          pl.BlockSpec((d, dh), lambda i: (0, 0)),
          pl.BlockSpec((dh,), lambda i: (0,)),
          pl.BlockSpec((dh, do), lambda i: (0, 0)),
          pl.BlockSpec((do,), lambda i: (0,)),
      ],
      out_specs=pl.BlockSpec((num_graphs, do), lambda i: (0, 0)),
      out_shape=jax.ShapeDtypeStruct((num_graphs, do), jnp.float32),
      scratch_shapes=[pltpu.VMEM((num_graphs, d), jnp.float32)],
  )(*xr_chunks, batch2d, Wf1, bf1, Wf2, bf2)


def kernel(x, edge_index, batch, W1, b1, W2, b2, W3, b3, Wmu, bmu, Wlv, blv,
           Wf1, bf1, Wf2, bf2):
  n, d = x.shape
  e = edge_index.shape[1]
  num_graphs = 64
  npad = -(-n // ROW_BLK) * ROW_BLK
  npad += -npad % (NS * 8)

  xp = jnp.pad(x, ((0, npad - n), (0, 0)))
  src = edge_index[0].reshape(NS, -1, IB, EC)
  dst = edge_index[1].reshape(NS, -1, IB, EC)
  ones = jnp.ones((npad // NS, 8), jnp.float32)
  zeros8 = jnp.zeros((npad // NS, 8), jnp.float32)
  zeros = jnp.zeros((RB, CC), jnp.float32)
  batch_p = jnp.pad(batch, (0, npad - n), constant_values=num_graphs)
  batch2d = jnp.broadcast_to(batch_p[:, None], (npad, 8))
  eps = jax.random.normal(jax.random.key(42), (n, Wmu.shape[1]),
                          dtype=jnp.float32)
  eps_p = jnp.pad(eps, ((0, npad - n), (0, 0)))

  aggf = _make_agg(npad, e)

  def agg_all(chunks):
    outs = []
    for chk in chunks:
      outs.extend(aggf(src, dst, zeros, chk))
    return outs

  deg0, deg1 = _make_deg(npad, e)(dst, ones, zeros8)
  xs0 = _prep(xp, deg0, deg1)
  s0 = agg_all(xs0)
  h1 = _layer(s0, deg0, deg1, W1, b1, relu=True, emit_raw=False)
  s1 = agg_all(h1)
  h2 = _layer(s1, deg0, deg1, W2, b2, relu=True, emit_raw=False)
  s2 = agg_all(h2)
  xs3, xr = _layer(s2, deg0, deg1, W3, b3, relu=True, emit_raw=True)
  s3 = agg_all(xs3)
  z, mu, lv = _final(s3, deg0, deg1, Wmu, bmu, Wlv, blv, eps_p)
  pm = _head(xr, batch2d, Wf1, bf1, Wf2, bf2, num_graphs)
  return (z[:n], mu[:n], lv[:n], pm)


# speedup vs baseline: 14.5668x; 1.0185x over previous
import functools

import jax
import jax.numpy as jnp
from jax import lax
from jax.experimental import pallas as pl
from jax.experimental.pallas import tpu as pltpu
from jax.experimental.pallas import tpu_sc as plsc

NC = 2
NS = 16
CC = 128
EC = 80
RB = 32
ROW_BLK = 1024


def _sc_mesh():
  return plsc.VectorSubcoreMesh(core_axis_name="c", subcore_axis_name="s",
                                num_cores=NC, num_subcores=NS)


IB = 25


@functools.cache
def _make_agg(n, e):
  ept = e // NS
  nw = ept // (IB * EC)
  nwc = nw // NC
  rpt = n // NS
  nr = rpt // RB

  out_type = [jax.ShapeDtypeStruct((n, CC), jnp.float32) for _ in range(2)]
  scratch = [
      pltpu.VMEM((IB, EC), jnp.int32),
      pltpu.VMEM((IB, EC), jnp.int32),
      pltpu.VMEM((EC, CC), jnp.float32),
      pltpu.VMEM((EC, CC), jnp.float32),
      pltpu.VMEM((RB, CC), jnp.float32),
      pltpu.VMEM_SHARED((n, CC), jnp.float32),
      pltpu.SemaphoreType.DMA,
      pltpu.SemaphoreType.DMA,
      pltpu.SemaphoreType.DMA,
      pltpu.SemaphoreType.DMA,
  ]

  @functools.partial(pl.kernel, out_type=out_type, mesh=_sc_mesh(),
                     scratch_types=scratch,
                     compiler_params=pltpu.CompilerParams(
                         use_tc_tiling_on_sc=False))
  def agg(src_hbm, dst_hbm, zeros_hbm, xs, out0, out1,
          src_v, dst_v, rows0, rows1, bounce, accum, sem0, sem1, ssem0,
          ssem1):
    cid = lax.axis_index("c")
    tid = lax.axis_index("s")
    row0 = tid * rpt
    for c in range(NC):
      out = out0 if c == 0 else out1

      @pl.when(cid == c)
      def _(c=c, out=out):
        for r in range(nr):
          if c == 0:
            pltpu.sync_copy(xs.at[pl.ds(row0 + r * RB, RB)], bounce)
          else:
            pltpu.sync_copy(zeros_hbm, bounce)
          pltpu.sync_copy(bounce, accum.at[pl.ds(row0 + r * RB, RB)])
        plsc.subcore_barrier()

        def outer(jo, carry):
          pltpu.sync_copy(src_hbm.at[tid, jo], src_v)
          pltpu.sync_copy(dst_hbm.at[tid, jo], dst_v)

          def body(jj, carry2):
            g0 = pltpu.async_copy(xs.at[src_v.at[2 * jj]], rows0, sem0)
            g1 = pltpu.async_copy(xs.at[src_v.at[2 * jj + 1]], rows1, sem1)
            g0.wait()
            s0 = pltpu.async_copy(rows0, accum.at[dst_v.at[2 * jj]], ssem0,
                                  add=True)
            g1.wait()
            s1 = pltpu.async_copy(rows1, accum.at[dst_v.at[2 * jj + 1]],
                                  ssem1, add=True)
            s0.wait()
            s1.wait()
            return carry2

          lax.fori_loop(0, IB // 2, body, 0)
          pltpu.async_copy(xs.at[src_v.at[IB - 1]], rows0, sem0).wait()
          pltpu.sync_copy(rows0, accum.at[dst_v.at[IB - 1]], add=True)
          return carry

        lax.fori_loop(c * nwc, (c + 1) * nwc, outer, 0)
        plsc.subcore_barrier()
        for r in range(nr):
          pltpu.sync_copy(accum.at[pl.ds(row0 + r * RB, RB)], bounce)
          pltpu.sync_copy(bounce, out.at[pl.ds(row0 + r * RB, RB)])
        plsc.subcore_barrier()

  return agg


def _make_deg(n, e):
  ept = e // NS
  nw = ept // (IB * EC)
  nwc = nw // NC
  rpt = n // NS

  out_type = [jax.ShapeDtypeStruct((n, 8), jnp.float32) for _ in range(NC)]
  scratch = [
      pltpu.VMEM((IB, EC), jnp.int32),
      pltpu.VMEM((rpt, 8), jnp.float32),
      pltpu.VMEM((EC, 8), jnp.float32),
      pltpu.VMEM_SHARED((n, 8), jnp.float32),
  ]

  @functools.partial(pl.kernel, out_type=out_type, mesh=_sc_mesh(),
                     scratch_types=scratch,
                     compiler_params=pltpu.CompilerParams(
                         use_tc_tiling_on_sc=False))
  def deg(dst_hbm, ones_hbm, zeros_hbm, out0, out1, dst_v, bounce, ones_v,
          accum):
    cid = lax.axis_index("c")
    tid = lax.axis_index("s")
    row0 = tid * rpt
    pltpu.sync_copy(ones_hbm.at[pl.ds(0, EC)], ones_v)
    for c in range(NC):

      @pl.when(cid == c)
      def _(c=c):
        if c == 0:
          pltpu.sync_copy(ones_hbm, bounce)
        else:
          pltpu.sync_copy(zeros_hbm, bounce)
        pltpu.sync_copy(bounce, accum.at[pl.ds(row0, rpt)])
        plsc.subcore_barrier()

        def outer(jo, carry):
          pltpu.sync_copy(dst_hbm.at[tid, jo], dst_v)

          def body(j, carry2):
            pltpu.sync_copy(ones_v, accum.at[dst_v.at[j]], add=True)
            return carry2

          lax.fori_loop(0, IB, body, 0)
          return carry

        lax.fori_loop(c * nwc, (c + 1) * nwc, outer, 0)
        plsc.subcore_barrier()
        pltpu.sync_copy(accum.at[pl.ds(row0, rpt)], bounce)
        pltpu.sync_copy(bounce, (out0 if c == 0 else out1).at[
            pl.ds(row0, rpt)])

  return deg


def _chunk_specs(nchunks):
  return [pl.BlockSpec((ROW_BLK, CC), lambda i: (i, 0))
          for _ in range(nchunks)]


def _prep(x, deg0, deg1):
  n, d = x.shape
  nc = d // CC

  def body(x_ref, d0_ref, d1_ref, *outs):
    dinv = lax.rsqrt(d0_ref[:, :1] + d1_ref[:, :1])
    xs = x_ref[...] * dinv
    for k in range(nc):
      outs[k][...] = xs[:, k * CC:(k + 1) * CC]

  return pl.pallas_call(
      body, grid=(n // ROW_BLK,),
      in_specs=[pl.BlockSpec((ROW_BLK, d), lambda i: (i, 0)),
                pl.BlockSpec((ROW_BLK, 8), lambda i: (i, 0)),
                pl.BlockSpec((ROW_BLK, 8), lambda i: (i, 0))],
      out_specs=_chunk_specs(nc),
      out_shape=[jax.ShapeDtypeStruct((n, CC), jnp.float32)] * nc,
  )(x, deg0, deg1)


def _layer(pairs, deg0, deg1, W, b, relu, emit_raw):
  n = pairs[0].shape[0]
  win, wout = W.shape
  nci, nco = win // CC, wout // CC

  def body(*refs):
    cr = refs[:2 * nci]
    d0_ref, d1_ref, w_ref, b_ref = refs[2 * nci:2 * nci + 4]
    outs = refs[2 * nci + 4:]
    dinv = lax.rsqrt(d0_ref[:, :1] + d1_ref[:, :1])
    s = jnp.concatenate(
        [cr[2 * k][...] + cr[2 * k + 1][...] for k in range(nci)],
        axis=1) * dinv
    h = jnp.dot(s, w_ref[...], preferred_element_type=jnp.float32) + b_ref[...]
    if relu:
      h = jnp.maximum(h, 0.0)
    hs = h * dinv
    for k in range(nco):
      outs[k][...] = hs[:, k * CC:(k + 1) * CC]
    if emit_raw:
      for k in range(nco):
        outs[nco + k][...] = h[:, k * CC:(k + 1) * CC]

  nout = nco * (2 if emit_raw else 1)
  res = pl.pallas_call(
      body, grid=(n // ROW_BLK,),
      in_specs=_chunk_specs(2 * nci) + [
          pl.BlockSpec((ROW_BLK, 8), lambda i: (i, 0)),
          pl.BlockSpec((ROW_BLK, 8), lambda i: (i, 0)),
          pl.BlockSpec((win, wout), lambda i: (0, 0)),
          pl.BlockSpec((wout,), lambda i: (0,)),
      ],
      out_specs=_chunk_specs(nout),
      out_shape=[jax.ShapeDtypeStruct((n, CC), jnp.float32)] * nout,
  )(*pairs, deg0, deg1, W, b)
  if emit_raw:
    return res[:nco], res[nco:]
  return res


def _final(pairs, deg0, deg1, Wmu, bmu, Wlv, blv, eps):
  n = pairs[0].shape[0]
  win, wout = Wmu.shape
  nci = win // CC

  def body(*refs):
    cr = refs[:2 * nci]
    (d0_ref, d1_ref, wmu_ref, bmu_ref, wlv_ref, blv_ref,
     eps_ref) = refs[2 * nci:2 * nci + 7]
    z_ref, mu_ref, lv_ref = refs[2 * nci + 7:]
    dinv = lax.rsqrt(d0_ref[:, :1] + d1_ref[:, :1])
    s = jnp.concatenate(
        [cr[2 * k][...] + cr[2 * k + 1][...] for k in range(nci)],
        axis=1) * dinv
    mu = jnp.dot(s, wmu_ref[...],
                 preferred_element_type=jnp.float32) + bmu_ref[...]
    lv = jnp.dot(s, wlv_ref[...],
                 preferred_element_type=jnp.float32) + blv_ref[...]
    mu_ref[...] = mu
    lv_ref[...] = lv
    z_ref[...] = mu + eps_ref[...] * jnp.exp(0.5 * lv)

  return pl.pallas_call(
      body, grid=(n // ROW_BLK,),
      in_specs=_chunk_specs(2 * nci) + [
          pl.BlockSpec((ROW_BLK, 8), lambda i: (i, 0)),
          pl.BlockSpec((ROW_BLK, 8), lambda i: (i, 0)),
          pl.BlockSpec((win, wout), lambda i: (0, 0)),
          pl.BlockSpec((wout,), lambda i: (0,)),
          pl.BlockSpec((win, wout), lambda i: (0, 0)),
          pl.BlockSpec((wout,), lambda i: (0,)),
          pl.BlockSpec((ROW_BLK, wout), lambda i: (i, 0)),
      ],
      out_specs=[pl.BlockSpec((ROW_BLK, wout), lambda i: (i, 0))] * 3,
      out_shape=[jax.ShapeDtypeStruct((n, wout), jnp.float32)] * 3,
  )(*pairs, deg0, deg1, Wmu, bmu, Wlv, blv, eps)


def _head(xr_chunks, batch2d, Wf1, bf1, Wf2, bf2, num_graphs):
  n = xr_chunks[0].shape[0]
  nci = len(xr_chunks)
  d = nci * CC
  dh = Wf1.shape[1]
  do = Wf2.shape[1]
  nsteps = n // ROW_BLK

  def body(*refs):
    cr = refs[:nci]
    b_ref, w1_ref, b1_ref, w2_ref, b2_ref = refs[nci:nci + 5]
    out_ref = refs[nci + 5]
    acc = refs[nci + 6]
    i = pl.program_id(0)

    @pl.when(i == 0)
    def _():
      acc[...] = jnp.full((num_graphs, d), -jnp.inf, jnp.float32)

    xr = jnp.concatenate([r[...] for r in cr], axis=1)
    bid = b_ref[:, :1]
    for g in range(num_graphs):
      m = jnp.max(jnp.where(bid == g, xr, -jnp.inf), axis=0, keepdims=True)
      acc[g:g + 1, :] = jnp.maximum(acc[g:g + 1, :], m)

    @pl.when(i == nsteps - 1)
    def _():
      x2 = acc[...]
      h = jnp.maximum(
          jnp.dot(x2, w1_ref[...], preferred_element_type=jnp.float32)
          + b1_ref[...], 0.0)
      out_ref[...] = (
          jnp.dot(h, w2_ref[...], preferred_element_type=jnp.float32)
          + b2_ref[...])

  return pl.pallas_call(
      body, grid=(nsteps,),
      in_specs=_chunk_specs(nci) + [
          pl.BlockSpec((ROW_BLK, 8), lambda i: (i, 0)),
          pl.BlockSpec((d, dh), lambda i: (0, 0)),
          pl.BlockSpec((dh,), lambda i: (0,)),
          pl.BlockSpec((dh, do), lambda i: (0, 0)),
          pl.BlockSpec((do,), lambda i: (0,)),
      ],
      out_specs=pl.BlockSpec((num_graphs, do), lambda i: (0, 0)),
      out_shape=jax.ShapeDtypeStruct((num_graphs, do), jnp.float32),
      scratch_shapes=[pltpu.VMEM((num_graphs, d), jnp.float32)],
  )(*xr_chunks, batch2d, Wf1, bf1, Wf2, bf2)


def kernel(x, edge_index, batch, W1, b1, W2, b2, W3, b3, Wmu, bmu, Wlv, blv,
           Wf1, bf1, Wf2, bf2):
  n, d = x.shape
  e = edge_index.shape[1]
  num_graphs = 64
  npad = -(-n // ROW_BLK) * ROW_BLK
  npad += -npad % (NS * 8)

  xp = jnp.pad(x, ((0, npad - n), (0, 0)))
  src = edge_index[0].reshape(NS, -1, IB, EC)
  dst = edge_index[1].reshape(NS, -1, IB, EC)
  ones = jnp.ones((npad // NS, 8), jnp.float32)
  zeros8 = jnp.zeros((npad // NS, 8), jnp.float32)
  zeros = jnp.zeros((RB, CC), jnp.float32)
  batch_p = jnp.pad(batch, (0, npad - n), constant_values=num_graphs)
  batch2d = jnp.broadcast_to(batch_p[:, None], (npad, 8))
  eps = jax.random.normal(jax.random.key(42), (n, Wmu.shape[1]),
                          dtype=jnp.float32)
  eps_p = jnp.pad(eps, ((0, npad - n), (0, 0)))

  aggf = _make_agg(npad, e)

  def agg_all(chunks):
    outs = []
    for chk in chunks:
      outs.extend(aggf(src, dst, zeros, chk))
    return outs

  deg0, deg1 = _make_deg(npad, e)(dst, ones, zeros8)
  xs0 = _prep(xp, deg0, deg1)
  s0 = agg_all(xs0)
  h1 = _layer(s0, deg0, deg1, W1, b1, relu=True, emit_raw=False)
  s1 = agg_all(h1)
  h2 = _layer(s1, deg0, deg1, W2, b2, relu=True, emit_raw=False)
  s2 = agg_all(h2)
  xs3, xr = _layer(s2, deg0, deg1, W3, b3, relu=True, emit_raw=True)
  s3 = agg_all(xs3)
  z, mu, lv = _final(s3, deg0, deg1, Wmu, bmu, Wlv, blv, eps_p)
  pm = _head(xr, batch2d, Wf1, bf1, Wf2, bf2, num_graphs)
  return (z[:n], mu[:n], lv[:n], pm)


# speedup vs baseline: 15.0107x; 1.0305x over previous
import functools

import jax
import jax.numpy as jnp
from jax import lax
from jax.experimental import pallas as pl
from jax.experimental.pallas import tpu as pltpu
from jax.experimental.pallas import tpu_sc as plsc

NC = 2
NS = 16
CC = 128
EC = 80
RB = 32
ROW_BLK = 1024


def _sc_mesh():
  return plsc.VectorSubcoreMesh(core_axis_name="c", subcore_axis_name="s",
                                num_cores=NC, num_subcores=NS)


IB = 25


@functools.cache
def _make_agg(n, e):
  ept = e // NS
  nw = ept // (IB * EC)
  nwc = nw // NC
  rpt = n // NS
  nr = rpt // RB

  out_type = [jax.ShapeDtypeStruct((n, CC), jnp.float32) for _ in range(2)]
  scratch = [
      pltpu.VMEM((IB, EC), jnp.int32),
      pltpu.VMEM((IB, EC), jnp.int32),
      pltpu.VMEM((EC, CC), jnp.float32),
      pltpu.VMEM((EC, CC), jnp.float32),
      pltpu.VMEM((EC, CC), jnp.float32),
      pltpu.VMEM((RB, CC), jnp.float32),
      pltpu.VMEM_SHARED((n, CC), jnp.float32),
  ] + [pltpu.SemaphoreType.DMA] * 6

  @functools.partial(pl.kernel, out_type=out_type, mesh=_sc_mesh(),
                     scratch_types=scratch,
                     compiler_params=pltpu.CompilerParams(
                         use_tc_tiling_on_sc=False))
  def agg(src_hbm, dst_hbm, zeros_hbm, xs, out0, out1,
          src_v, dst_v, rows0, rows1, rows2, bounce, accum, sem0, sem1,
          sem2, ssem0, ssem1, ssem2):
    cid = lax.axis_index("c")
    tid = lax.axis_index("s")
    row0 = tid * rpt
    for c in range(NC):
      out = out0 if c == 0 else out1

      @pl.when(cid == c)
      def _(c=c, out=out):
        for r in range(nr):
          if c == 0:
            pltpu.sync_copy(xs.at[pl.ds(row0 + r * RB, RB)], bounce)
          else:
            pltpu.sync_copy(zeros_hbm, bounce)
          pltpu.sync_copy(bounce, accum.at[pl.ds(row0 + r * RB, RB)])
        plsc.subcore_barrier()

        def outer(jo, carry):
          pltpu.sync_copy(src_hbm.at[tid, jo], src_v)
          pltpu.sync_copy(dst_hbm.at[tid, jo], dst_v)

          def body(jt, carry2):
            j = 3 * jt
            g0 = pltpu.async_copy(xs.at[src_v.at[j]], rows0, sem0)
            g1 = pltpu.async_copy(xs.at[src_v.at[j + 1]], rows1, sem1)
            g2 = pltpu.async_copy(xs.at[src_v.at[j + 2]], rows2, sem2)
            g0.wait()
            s0 = pltpu.async_copy(rows0, accum.at[dst_v.at[j]], ssem0,
                                  add=True)
            g1.wait()
            s1 = pltpu.async_copy(rows1, accum.at[dst_v.at[j + 1]], ssem1,
                                  add=True)
            g2.wait()
            s2 = pltpu.async_copy(rows2, accum.at[dst_v.at[j + 2]], ssem2,
                                  add=True)
            s0.wait()
            s1.wait()
            s2.wait()
            return carry2

          lax.fori_loop(0, IB // 3, body, 0)
          pltpu.async_copy(xs.at[src_v.at[IB - 1]], rows0, sem0).wait()
          pltpu.sync_copy(rows0, accum.at[dst_v.at[IB - 1]], add=True)
          return carry

        lax.fori_loop(c * nwc, (c + 1) * nwc, outer, 0)
        plsc.subcore_barrier()
        for r in range(nr):
          pltpu.sync_copy(accum.at[pl.ds(row0 + r * RB, RB)], bounce)
          pltpu.sync_copy(bounce, out.at[pl.ds(row0 + r * RB, RB)])
        plsc.subcore_barrier()

  return agg


def _make_deg(n, e):
  ept = e // NS
  nw = ept // (IB * EC)
  nwc = nw // NC
  rpt = n // NS

  out_type = [jax.ShapeDtypeStruct((n, 8), jnp.float32) for _ in range(NC)]
  scratch = [
      pltpu.VMEM((IB, EC), jnp.int32),
      pltpu.VMEM((EC, 8), jnp.float32),
      pltpu.VMEM_SHARED((n, 8), jnp.float32),
  ]

  @functools.partial(pl.kernel, out_type=out_type, mesh=_sc_mesh(),
                     scratch_types=scratch,
                     compiler_params=pltpu.CompilerParams(
                         use_tc_tiling_on_sc=False))
  def deg(dst_hbm, ones_hbm, zeros_hbm, out0, out1, dst_v, bounce, accum):
    cid = lax.axis_index("c")
    tid = lax.axis_index("s")
    row0 = tid * rpt
    nr8 = rpt // EC
    for c in range(NC):

      @pl.when(cid == c)
      def _(c=c):
        seed = ones_hbm if c == 0 else zeros_hbm
        for r in range(nr8):
          pltpu.sync_copy(seed.at[pl.ds(r * EC, EC)], bounce)
          pltpu.sync_copy(bounce, accum.at[pl.ds(row0 + r * EC, EC)])
        if c != 0:
          pltpu.sync_copy(ones_hbm.at[pl.ds(0, EC)], bounce)
        plsc.subcore_barrier()

        def outer(jo, carry):
          pltpu.sync_copy(dst_hbm.at[tid, jo], dst_v)

          def body(j, carry2):
            pltpu.sync_copy(bounce, accum.at[dst_v.at[j]], add=True)
            return carry2

          lax.fori_loop(0, IB, body, 0)
          return carry

        lax.fori_loop(c * nwc, (c + 1) * nwc, outer, 0)
        plsc.subcore_barrier()
        out = out0 if c == 0 else out1
        for r in range(nr8):
          pltpu.sync_copy(accum.at[pl.ds(row0 + r * EC, EC)], bounce)
          pltpu.sync_copy(bounce, out.at[pl.ds(row0 + r * EC, EC)])

  return deg


def _chunk_specs(nchunks):
  return [pl.BlockSpec((ROW_BLK, CC), lambda i: (i, 0))
          for _ in range(nchunks)]


def _prep(x, deg0, deg1):
  n, d = x.shape
  nc = d // CC

  def body(x_ref, d0_ref, d1_ref, *outs):
    dinv = lax.rsqrt(d0_ref[:, :1] + d1_ref[:, :1])
    xs = x_ref[...] * dinv
    for k in range(nc):
      outs[k][...] = xs[:, k * CC:(k + 1) * CC]

  return pl.pallas_call(
      body, grid=(n // ROW_BLK,),
      in_specs=[pl.BlockSpec((ROW_BLK, d), lambda i: (i, 0)),
                pl.BlockSpec((ROW_BLK, 8), lambda i: (i, 0)),
                pl.BlockSpec((ROW_BLK, 8), lambda i: (i, 0))],
      out_specs=_chunk_specs(nc),
      out_shape=[jax.ShapeDtypeStruct((n, CC), jnp.float32)] * nc,
  )(x, deg0, deg1)


def _layer(pairs, deg0, deg1, W, b, relu, emit_raw):
  n = pairs[0].shape[0]
  win, wout = W.shape
  nci, nco = win // CC, wout // CC

  def body(*refs):
    cr = refs[:2 * nci]
    d0_ref, d1_ref, w_ref, b_ref = refs[2 * nci:2 * nci + 4]
    outs = refs[2 * nci + 4:]
    dinv = lax.rsqrt(d0_ref[:, :1] + d1_ref[:, :1])
    s = jnp.concatenate(
        [cr[2 * k][...] + cr[2 * k + 1][...] for k in range(nci)],
        axis=1) * dinv
    h = jnp.dot(s, w_ref[...], preferred_element_type=jnp.float32) + b_ref[...]
    if relu:
      h = jnp.maximum(h, 0.0)
    hs = h * dinv
    for k in range(nco):
      outs[k][...] = hs[:, k * CC:(k + 1) * CC]
    if emit_raw:
      for k in range(nco):
        outs[nco + k][...] = h[:, k * CC:(k + 1) * CC]

  nout = nco * (2 if emit_raw else 1)
  res = pl.pallas_call(
      body, grid=(n // ROW_BLK,),
      in_specs=_chunk_specs(2 * nci) + [
          pl.BlockSpec((ROW_BLK, 8), lambda i: (i, 0)),
          pl.BlockSpec((ROW_BLK, 8), lambda i: (i, 0)),
          pl.BlockSpec((win, wout), lambda i: (0, 0)),
          pl.BlockSpec((wout,), lambda i: (0,)),
      ],
      out_specs=_chunk_specs(nout),
      out_shape=[jax.ShapeDtypeStruct((n, CC), jnp.float32)] * nout,
  )(*pairs, deg0, deg1, W, b)
  if emit_raw:
    return res[:nco], res[nco:]
  return res


def _final(pairs, deg0, deg1, Wmu, bmu, Wlv, blv, eps):
  n = pairs[0].shape[0]
  win, wout = Wmu.shape
  nci = win // CC

  def body(*refs):
    cr = refs[:2 * nci]
    (d0_ref, d1_ref, wmu_ref, bmu_ref, wlv_ref, blv_ref,
     eps_ref) = refs[2 * nci:2 * nci + 7]
    z_ref, mu_ref, lv_ref = refs[2 * nci + 7:]
    dinv = lax.rsqrt(d0_ref[:, :1] + d1_ref[:, :1])
    s = jnp.concatenate(
        [cr[2 * k][...] + cr[2 * k + 1][...] for k in range(nci)],
        axis=1) * dinv
    mu = jnp.dot(s, wmu_ref[...],
                 preferred_element_type=jnp.float32) + bmu_ref[...]
    lv = jnp.dot(s, wlv_ref[...],
                 preferred_element_type=jnp.float32) + blv_ref[...]
    mu_ref[...] = mu
    lv_ref[...] = lv
    z_ref[...] = mu + eps_ref[...] * jnp.exp(0.5 * lv)

  return pl.pallas_call(
      body, grid=(n // ROW_BLK,),
      in_specs=_chunk_specs(2 * nci) + [
          pl.BlockSpec((ROW_BLK, 8), lambda i: (i, 0)),
          pl.BlockSpec((ROW_BLK, 8), lambda i: (i, 0)),
          pl.BlockSpec((win, wout), lambda i: (0, 0)),
          pl.BlockSpec((wout,), lambda i: (0,)),
          pl.BlockSpec((win, wout), lambda i: (0, 0)),
          pl.BlockSpec((wout,), lambda i: (0,)),
          pl.BlockSpec((ROW_BLK, wout), lambda i: (i, 0)),
      ],
      out_specs=[pl.BlockSpec((ROW_BLK, wout), lambda i: (i, 0))] * 3,
      out_shape=[jax.ShapeDtypeStruct((n, wout), jnp.float32)] * 3,
  )(*pairs, deg0, deg1, Wmu, bmu, Wlv, blv, eps)


def _head(xr_chunks, batch2d, Wf1, bf1, Wf2, bf2, num_graphs):
  n = xr_chunks[0].shape[0]
  nci = len(xr_chunks)
  d = nci * CC
  dh = Wf1.shape[1]
  do = Wf2.shape[1]
  nsteps = n // ROW_BLK

  def body(*refs):
    cr = refs[:nci]
    b_ref, w1_ref, b1_ref, w2_ref, b2_ref = refs[nci:nci + 5]
    out_ref = refs[nci + 5]
    acc = refs[nci + 6]
    i = pl.program_id(0)

    @pl.when(i == 0)
    def _():
      acc[...] = jnp.full((num_graphs, d), -jnp.inf, jnp.float32)

    xr = jnp.concatenate([r[...] for r in cr], axis=1)
    bid = b_ref[:, :1]
    for g in range(num_graphs):
      m = jnp.max(jnp.where(bid == g, xr, -jnp.inf), axis=0, keepdims=True)
      acc[g:g + 1, :] = jnp.maximum(acc[g:g + 1, :], m)

    @pl.when(i == nsteps - 1)
    def _():
      x2 = acc[...]
      h = jnp.maximum(
          jnp.dot(x2, w1_ref[...], preferred_element_type=jnp.float32)
          + b1_ref[...], 0.0)
      out_ref[...] = (
          jnp.dot(h, w2_ref[...], preferred_element_type=jnp.float32)
          + b2_ref[...])

  return pl.pallas_call(
      body, grid=(nsteps,),
      in_specs=_chunk_specs(nci) + [
          pl.BlockSpec((ROW_BLK, 8), lambda i: (i, 0)),
          pl.BlockSpec((d, dh), lambda i: (0, 0)),
          pl.BlockSpec((dh,), lambda i: (0,)),
          pl.BlockSpec((dh, do), lambda i: (0, 0)),
          pl.BlockSpec((do,), lambda i: (0,)),
      ],
      out_specs=pl.BlockSpec((num_graphs, do), lambda i: (0, 0)),
      out_shape=jax.ShapeDtypeStruct((num_graphs, do), jnp.float32),
      scratch_shapes=[pltpu.VMEM((num_graphs, d), jnp.float32)],
  )(*xr_chunks, batch2d, Wf1, bf1, Wf2, bf2)


def kernel(x, edge_index, batch, W1, b1, W2, b2, W3, b3, Wmu, bmu, Wlv, blv,
           Wf1, bf1, Wf2, bf2):
  n, d = x.shape
  e = edge_index.shape[1]
  num_graphs = 64
  npad = -(-n // ROW_BLK) * ROW_BLK
  npad += -npad % (NS * 8)

  xp = jnp.pad(x, ((0, npad - n), (0, 0)))
  src = edge_index[0].reshape(NS, -1, IB, EC)
  dst = edge_index[1].reshape(NS, -1, IB, EC)
  ones = jnp.ones((npad // NS, 8), jnp.float32)
  zeros8 = jnp.zeros((npad // NS, 8), jnp.float32)
  zeros = jnp.zeros((RB, CC), jnp.float32)
  batch_p = jnp.pad(batch, (0, npad - n), constant_values=num_graphs)
  batch2d = jnp.broadcast_to(batch_p[:, None], (npad, 8))
  eps = jax.random.normal(jax.random.key(42), (n, Wmu.shape[1]),
                          dtype=jnp.float32)
  eps_p = jnp.pad(eps, ((0, npad - n), (0, 0)))

  aggf = _make_agg(npad, e)

  def agg_all(chunks):
    outs = []
    for chk in chunks:
      outs.extend(aggf(src, dst, zeros, chk))
    return outs

  deg0, deg1 = _make_deg(npad, e)(dst, ones, zeros8)
  xs0 = _prep(xp, deg0, deg1)
  s0 = agg_all(xs0)
  h1 = _layer(s0, deg0, deg1, W1, b1, relu=True, emit_raw=False)
  s1 = agg_all(h1)
  h2 = _layer(s1, deg0, deg1, W2, b2, relu=True, emit_raw=False)
  s2 = agg_all(h2)
  xs3, xr = _layer(s2, deg0, deg1, W3, b3, relu=True, emit_raw=True)
  s3 = agg_all(xs3)
  z, mu, lv = _final(s3, deg0, deg1, Wmu, bmu, Wlv, blv, eps_p)
  pm = _head(xr, batch2d, Wf1, bf1, Wf2, bf2, num_graphs)
  return (z[:n], mu[:n], lv[:n], pm)
